# baseline (speedup 1.0000x reference)
import functools

import jax
import jax.numpy as jnp
from jax import lax
from jax.experimental import pallas as pl
from jax.experimental.pallas import tpu as pltpu

N_DOCS = 4096
D = 384
H = 256
KR = 50
K = 5
PAD = 64
BIG = 1 << 30
NEG = float("-inf")


def _rag_kernel(q_row_ref, q_col_ref, corpus_ref, w1_ref, b1_ref, w2_ref,
                b2_ref, koff_ref, kroff_ref,
                ridx_ref, isc_ref, fidx_ref, fsc_ref):
    f32 = jnp.float32
    hi = jax.lax.Precision.HIGHEST

    q = q_row_ref[...]
    qn_inv = 1.0 / jnp.clip(jnp.sqrt(jnp.sum(q * q)), 1e-12)
    qn_row = q * qn_inv
    qn_col = q_col_ref[...] * qn_inv

    c = corpus_ref[...]
    c_inv = 1.0 / jnp.clip(jnp.sqrt(jnp.sum(c * c, axis=1, keepdims=True)),
                           1e-12)
    cn = c * c_inv

    sims_row = lax.dot_general(qn_row, cn, (((1,), (1,)), ((), ())),
                               precision=hi)
    sims_col = jnp.dot(cn, qn_col, precision=hi)

    col_n = lax.broadcasted_iota(jnp.int32, (1, N_DOCS), 1)
    row_p = lax.broadcasted_iota(jnp.int32, (PAD, 1), 0)
    colpn = lax.broadcasted_iota(jnp.int32, (PAD, N_DOCS), 1)

    def topk_body(i, carry):
        masked, ridx = carry
        m = jnp.max(masked)
        idx = jnp.min(jnp.where(masked == m, col_n, BIG))
        ridx = jnp.where(row_p == i, idx, ridx)
        masked = jnp.where(col_n == idx, NEG, masked)
        return masked, ridx

    ridx0 = jnp.zeros((PAD, 1), jnp.int32)
    _, ridx = lax.fori_loop(0, KR, topk_body, (sims_row, ridx0))

    koff = koff_ref[...]
    kroff = kroff_ref[...]

    ish50 = jnp.clip(ridx + kroff, 0, N_DOCS - 1)
    ish5 = jnp.clip(ridx + koff, 0, N_DOCS - 1)

    oh50 = jnp.where((colpn == ish50) & (row_p < KR), f32(1), f32(0))
    oh5 = jnp.where((colpn == ish5) & (row_p < K), f32(1), f32(0))
    docs = jnp.dot(oh50, cn, precision=hi)
    rsc = jnp.dot(oh50, sims_col, precision=hi)
    isc_ref[...] = jnp.dot(oh5, sims_col, precision=hi)

    qh = jnp.dot(qn_row, w1_ref[:D, :], precision=hi) + b1_ref[...]
    h = jnp.tanh(jnp.dot(docs, w1_ref[D:, :], precision=hi) + qh)
    cross = jnp.dot(h, w2_ref[...], precision=hi) + b2_ref[...]
    rer = cross + 0.1 * rsc
    rer = jnp.where(row_p < KR, rer, NEG)

    out_idx = ridx + kroff

    def final_body(i, carry):
        rmask, fidx, fsc = carry
        m = jnp.max(rmask)
        pos = jnp.min(jnp.where(rmask == m, row_p, BIG))
        g = jnp.sum(jnp.where(row_p == pos, out_idx, 0))
        fidx = jnp.where(row_p == i, g, fidx)
        fsc = jnp.where(row_p == i, m, fsc)
        rmask = jnp.where(row_p == pos, NEG, rmask)
        return rmask, fidx, fsc

    fidx0 = jnp.zeros((PAD, 1), jnp.int32)
    fsc0 = jnp.zeros((PAD, 1), f32)
    _, fidx, fsc = lax.fori_loop(0, K, final_body, (rer, fidx0, fsc0))

    ridx_ref[...] = ridx
    fidx_ref[...] = fidx
    fsc_ref[...] = fsc


@jax.jit
def _run(query_embed, corpus_embeds, W1, b1, W2, b2, koff, kroff):
    q_col = query_embed.reshape(D, 1)
    b1r = b1.reshape(1, H)
    b2r = b2.reshape(1, 1)
    koff_v = jnp.full((PAD, 1), koff, jnp.int32)
    kroff_v = jnp.full((PAD, 1), kroff, jnp.int32)
    out_shape = [
        jax.ShapeDtypeStruct((PAD, 1), jnp.int32),
        jax.ShapeDtypeStruct((PAD, 1), jnp.float32),
        jax.ShapeDtypeStruct((PAD, 1), jnp.int32),
        jax.ShapeDtypeStruct((PAD, 1), jnp.float32),
    ]
    ridx, isc, fidx, fsc = pl.pallas_call(
        _rag_kernel,
        out_shape=out_shape,
    )(query_embed, q_col, corpus_embeds, W1, b1r, W2, b2r, koff_v, kroff_v)

    final_idx = fidx[:K, 0][None, :]
    final_sc = fsc[:K, 0][None, :]
    init_idx = (ridx[:K, 0] + koff)[None, :].astype(jnp.int32)
    init_sc = isc[:K, 0][None, :]
    return final_idx, final_sc, init_idx, init_sc


def kernel(query_embed, corpus_embeds, W1, b1, W2, b2, k, k_retrieval):
    koff = jnp.asarray(k, jnp.int32) - K
    kroff = jnp.asarray(k_retrieval, jnp.int32) - KR
    return _run(query_embed, corpus_embeds, W1, b1, W2, b2, koff, kroff)


# speedup vs baseline: 1.1952x; 1.1952x over previous
import functools

import jax
import jax.numpy as jnp
from jax import lax
from jax.experimental import pallas as pl
from jax.experimental.pallas import tpu as pltpu

N_DOCS = 4096
D = 384
H = 256
KR = 50
K = 5
PAD = 64
BIG = 1 << 30
NEG = float("-inf")


def _rag_kernel(q_row_ref, q_col_ref, corpus_ref, w1_ref, b1_ref, w2_ref,
                b2_ref, koff_ref, kroff_ref,
                ridx_ref, isc_ref, fidx_ref, fsc_ref):
    f32 = jnp.float32
    hi = jax.lax.Precision.HIGHEST

    q = q_row_ref[...]
    qn_inv = 1.0 / jnp.clip(jnp.sqrt(jnp.sum(q * q)), 1e-12)
    qn_row = q * qn_inv
    qn_col = q_col_ref[...] * qn_inv

    c = corpus_ref[...]
    c_inv = 1.0 / jnp.clip(jnp.sqrt(jnp.sum(c * c, axis=1, keepdims=True)),
                           1e-12)
    cn = c * c_inv

    sims_row = lax.dot_general(qn_row, cn, (((1,), (1,)), ((), ())),
                               precision=hi)
    sims_col = jnp.dot(cn, qn_col, precision=hi)

    row_p = lax.broadcasted_iota(jnp.int32, (PAD, 1), 0)
    colpn = lax.broadcasted_iota(jnp.int32, (PAD, N_DOCS), 1)

    NC = N_DOCS // 8
    sims8 = jnp.reshape(sims_row, (8, NC))
    pos8 = (lax.broadcasted_iota(jnp.int32, (8, NC), 0) * NC
            + lax.broadcasted_iota(jnp.int32, (8, NC), 1))

    def topk_body(i, carry):
        masked, ridx = carry
        m = jnp.max(masked)
        idx = jnp.min(jnp.where(masked == m, pos8, BIG))
        ridx = jnp.where(row_p == i, idx, ridx)
        masked = jnp.where(pos8 == idx, NEG, masked)
        return masked, ridx

    ridx0 = jnp.zeros((PAD, 1), jnp.int32)
    _, ridx = lax.fori_loop(0, KR, topk_body, (sims8, ridx0))

    koff = koff_ref[...]
    kroff = kroff_ref[...]

    ish50 = jnp.clip(ridx + kroff, 0, N_DOCS - 1)
    ish5 = jnp.clip(ridx + koff, 0, N_DOCS - 1)

    oh50 = jnp.where((colpn == ish50) & (row_p < KR), f32(1), f32(0))
    docs = jnp.dot(oh50, cn)

    ish5r = pltpu.roll(ish5, 56, 0)
    comb = jnp.where(row_p < KR, ish50, ish5r)
    vrow = (row_p < KR) | ((row_p >= 56) & (row_p < 56 + K))
    ohc = jnp.where((colpn == comb) & vrow, f32(1), f32(0))
    scg = jnp.dot(ohc, sims_col)
    rsc = jnp.where(row_p < KR, scg, f32(0))
    isc_ref[...] = pltpu.roll(scg, PAD - 56, 0)

    qh = jnp.dot(qn_row, w1_ref[:D, :]) + b1_ref[...]
    h = jnp.tanh(jnp.dot(docs, w1_ref[D:, :]) + qh)
    cross = jnp.dot(h, w2_ref[...]) + b2_ref[...]
    rer = cross + 0.1 * rsc
    rer = jnp.where(row_p < KR, rer, NEG)

    out_idx = ridx + kroff

    def final_body(i, carry):
        rmask, fidx, fsc = carry
        m = jnp.max(rmask)
        pos = jnp.min(jnp.where(rmask == m, row_p, BIG))
        g = jnp.sum(jnp.where(row_p == pos, out_idx, 0))
        fidx = jnp.where(row_p == i, g, fidx)
        fsc = jnp.where(row_p == i, m, fsc)
        rmask = jnp.where(row_p == pos, NEG, rmask)
        return rmask, fidx, fsc

    fidx0 = jnp.zeros((PAD, 1), jnp.int32)
    fsc0 = jnp.zeros((PAD, 1), f32)
    _, fidx, fsc = lax.fori_loop(0, K, final_body, (rer, fidx0, fsc0))

    ridx_ref[...] = ridx
    fidx_ref[...] = fidx
    fsc_ref[...] = fsc


@jax.jit
def _run(query_embed, corpus_embeds, W1, b1, W2, b2, koff, kroff):
    q_col = query_embed.reshape(D, 1)
    b1r = b1.reshape(1, H)
    b2r = b2.reshape(1, 1)
    koff_v = jnp.full((PAD, 1), koff, jnp.int32)
    kroff_v = jnp.full((PAD, 1), kroff, jnp.int32)
    out_shape = [
        jax.ShapeDtypeStruct((PAD, 1), jnp.int32),
        jax.ShapeDtypeStruct((PAD, 1), jnp.float32),
        jax.ShapeDtypeStruct((PAD, 1), jnp.int32),
        jax.ShapeDtypeStruct((PAD, 1), jnp.float32),
    ]
    ridx, isc, fidx, fsc = pl.pallas_call(
        _rag_kernel,
        out_shape=out_shape,
    )(query_embed, q_col, corpus_embeds, W1, b1r, W2, b2r, koff_v, kroff_v)

    final_idx = fidx[:K, 0][None, :]
    final_sc = fsc[:K, 0][None, :]
    init_idx = (ridx[:K, 0] + koff)[None, :].astype(jnp.int32)
    init_sc = isc[:K, 0][None, :]
    return final_idx, final_sc, init_idx, init_sc


def kernel(query_embed, corpus_embeds, W1, b1, W2, b2, k, k_retrieval):
    koff = jnp.asarray(k, jnp.int32) - K
    kroff = jnp.asarray(k_retrieval, jnp.int32) - KR
    return _run(query_embed, corpus_embeds, W1, b1, W2, b2, koff, kroff)
